# baseline (speedup 1.0000x reference)
import functools

import jax
import jax.numpy as jnp
from jax import lax
from jax.experimental import pallas as pl
from jax.experimental.pallas import tpu as pltpu
from jax.experimental.pallas import tpu_sc as plsc

_G = 128
_LANES = 16
_CHROWS = 64


def _sc_extract_fn(N, C, NC, NS):
    NW = NC * NS
    bpw = N // NW
    nch = bpw // _CHROWS

    mesh = plsc.VectorSubcoreMesh(core_axis_name="c", subcore_axis_name="s")

    @functools.partial(
        pl.kernel,
        out_type=jax.ShapeDtypeStruct((N // _G, _G), jnp.float32),
        mesh=mesh,
        scratch_types=[
            pltpu.VMEM((bpw,), jnp.int32),
            pltpu.VMEM((2, _CHROWS, C), jnp.float32),
            pltpu.VMEM((bpw // _G, _G), jnp.float32),
            pltpu.SemaphoreType.DMA((2,)),
        ],
        compiler_params=pltpu.CompilerParams(use_tc_tiling_on_sc=True,
                                             needs_layout_passes=False,
                                             skip_device_barrier=True),
    )
    def sc_extract(t_hbm, sim_hbm, simt_out, t_v, chunk_v, res_v, sems):
        wid = lax.axis_index("s") * NC + lax.axis_index("c")
        base = wid * bpw
        pltpu.sync_copy(t_hbm.at[pl.ds(base, bpw)], t_v)

        def cp(c):
            return pltpu.make_async_copy(
                sim_hbm.at[pl.ds(base + c * _CHROWS, _CHROWS), :],
                chunk_v.at[c % 2], sems.at[c % 2])

        cp(0).start()
        for c in range(nch):
            if c + 1 < nch:
                cp(c + 1).start()
            cp(c).wait()
            for j in range(_CHROWS // _LANES):
                r16 = lax.iota(jnp.int32, _LANES) + j * _LANES
                t16 = t_v[pl.ds(c * _CHROWS + j * _LANES, _LANES)]
                vals = plsc.load_gather(chunk_v.at[c % 2], [r16, t16])
                jj = c * (_CHROWS // _LANES) + j
                res_v[jj // (_G // _LANES),
                      pl.ds((jj % (_G // _LANES)) * _LANES, _LANES)] = vals
        pltpu.sync_copy(res_v, simt_out.at[pl.ds(wid * (bpw // _G),
                                                 bpw // _G)])

    return sc_extract


def _tc_loss_fn(N, C, R):
    def body(x_ref, t_ref, simt_ref, out_ref):
        i = pl.program_id(0)

        @pl.when(i == 0)
        def _init():
            out_ref[0, 0] = 0.0

        x = x_ref[...].reshape(R, _G, C)
        cols = lax.broadcasted_iota(jnp.int32, (R, _G, C), 2)
        oh = cols == t_ref[...][:, :, None]
        m = jnp.max(x, axis=2)
        s = jnp.sum(jnp.exp(x - m[:, :, None]), axis=2)
        lse = m + jnp.log(s)
        logit_t = jnp.sum(jnp.where(oh, x, 0.0), axis=2)
        w = 10.0 / (1.0 + jnp.exp(4.0 * simt_ref[...]))
        out_ref[0, 0] += jnp.sum(w * (lse - logit_t))

    return pl.pallas_call(
        body,
        grid=(N // (R * _G),),
        in_specs=[
            pl.BlockSpec((R * _G, C), lambda i: (i, 0)),
            pl.BlockSpec((R, _G), lambda i: (i, 0)),
            pl.BlockSpec((R, _G), lambda i: (i, 0)),
        ],
        out_specs=pl.BlockSpec(memory_space=pltpu.MemorySpace.SMEM),
        out_shape=jax.ShapeDtypeStruct((1, 1), jnp.float32),
        compiler_params=pltpu.CompilerParams(
            dimension_semantics=("arbitrary",)),
    )


def kernel(src_logits, hoi_text_similarity, target_classes_i):
    N, C = src_logits.shape
    t = target_classes_i.astype(jnp.int32)

    info = plsc.get_sparse_core_info()
    simt = _sc_extract_fn(N, C, info.num_cores, info.num_subcores)(
        t, hoi_text_similarity)

    R = 16
    out = _tc_loss_fn(N, C, R)(src_logits, t.reshape(N // _G, _G), simt)
    return out[0, 0] / N


# speedup vs baseline: 1.0036x; 1.0036x over previous
import functools

import jax
import jax.numpy as jnp
from jax import lax
from jax.experimental import pallas as pl
from jax.experimental.pallas import tpu as pltpu
from jax.experimental.pallas import tpu_sc as plsc

_G = 128
_LANES = 16
_CHROWS = 64


def _sc_extract_fn(N, C, NC, NS):
    NW = NC * NS
    bpw = N // NW
    nch = bpw // _CHROWS

    mesh = plsc.VectorSubcoreMesh(core_axis_name="c", subcore_axis_name="s")

    @functools.partial(
        pl.kernel,
        out_type=jax.ShapeDtypeStruct((N,), jnp.float32),
        mesh=mesh,
        scratch_types=[
            pltpu.VMEM((bpw,), jnp.int32),
            pltpu.VMEM((2, _CHROWS, C), jnp.float32),
            pltpu.VMEM((bpw,), jnp.float32),
            pltpu.SemaphoreType.DMA((2,)),
        ],
        compiler_params=pltpu.CompilerParams(use_tc_tiling_on_sc=True,
                                             needs_layout_passes=False),
    )
    def sc_extract(t_hbm, sim_hbm, simt_out, t_v, chunk_v, res_v, sems):
        wid = lax.axis_index("s") * NC + lax.axis_index("c")
        base = wid * bpw
        pltpu.sync_copy(t_hbm.at[pl.ds(base, bpw)], t_v)

        def cp(c):
            return pltpu.make_async_copy(
                sim_hbm.at[pl.ds(base + c * _CHROWS, _CHROWS), :],
                chunk_v.at[c % 2], sems.at[c % 2])

        cp(0).start()
        for c in range(nch):
            if c + 1 < nch:
                cp(c + 1).start()
            cp(c).wait()
            for j in range(_CHROWS // _LANES):
                r16 = lax.iota(jnp.int32, _LANES) + j * _LANES
                t16 = t_v[pl.ds(c * _CHROWS + j * _LANES, _LANES)]
                vals = plsc.load_gather(chunk_v.at[c % 2], [r16, t16])
                res_v[pl.ds((c * (_CHROWS // _LANES) + j) * _LANES,
                            _LANES)] = vals
        pltpu.sync_copy(res_v, simt_out.at[pl.ds(base, bpw)])

    return sc_extract


def _tc_loss_fn(N, C, R):
    def body(x_ref, t_ref, simt_ref, out_ref):
        i = pl.program_id(0)

        @pl.when(i == 0)
        def _init():
            out_ref[0, 0] = 0.0

        x = x_ref[...].reshape(R, _G, C)
        cols = lax.broadcasted_iota(jnp.int32, (R, _G, C), 2)
        oh = cols == t_ref[...][:, :, None]
        m = jnp.max(x, axis=2)
        s = jnp.sum(jnp.exp(x - m[:, :, None]), axis=2)
        lse = m + jnp.log(s)
        logit_t = jnp.sum(jnp.where(oh, x, 0.0), axis=2)
        w = 10.0 / (1.0 + jnp.exp(4.0 * simt_ref[...].reshape(R, _G)))
        out_ref[0, 0] += jnp.sum(w * (lse - logit_t))

    return pl.pallas_call(
        body,
        grid=(N // (R * _G),),
        in_specs=[
            pl.BlockSpec((R * _G, C), lambda i: (i, 0)),
            pl.BlockSpec((R, _G), lambda i: (i, 0)),
            pl.BlockSpec((R * _G,), lambda i: (i,)),
        ],
        out_specs=pl.BlockSpec(memory_space=pltpu.MemorySpace.SMEM),
        out_shape=jax.ShapeDtypeStruct((1, 1), jnp.float32),
        compiler_params=pltpu.CompilerParams(
            dimension_semantics=("arbitrary",)),
    )


def kernel(src_logits, hoi_text_similarity, target_classes_i):
    N, C = src_logits.shape
    t = target_classes_i.astype(jnp.int32)

    info = plsc.get_sparse_core_info()
    simt = _sc_extract_fn(N, C, info.num_cores, info.num_subcores)(
        t, hoi_text_similarity)

    R = 16
    out = _tc_loss_fn(N, C, R)(src_logits, t.reshape(N // _G, _G), simt)
    return out[0, 0] / N
